# baseline (speedup 1.0000x reference)
import jax
import jax.numpy as jnp
from jax import lax
from jax.experimental import pallas as pl
from jax.experimental.pallas import tpu as pltpu, tpu_sc as plsc

NC, NS, LANES = 2, 16, 16
NW = NC * NS
D = 64
L = 50
CB = 8
EPS = 1e-5
J = D // LANES


def _rsqrt_s(v):
    i = lax.bitcast_convert_type(v, jnp.int32)
    y = lax.bitcast_convert_type(jnp.int32(0x5F3759DF) - (i >> 1), jnp.float32)
    for _ in range(3):
        y = y * (1.5 - 0.5 * v * y * y)
    return y


def _body(x_hbm, table_hbm, w_hbm, b_hbm, out_hbm,
          idx0, idx1, ib0, ib1, ob0, ob1, w_v, b_v, si0, si1, so0, so1):
    B = x_hbm.shape[0]
    per_w = B // NW
    n_chunks = per_w // CB
    wid = lax.axis_index("s") * NC + lax.axis_index("c")
    b0 = wid * per_w
    pltpu.sync_copy(w_hbm, w_v)
    pltpu.sync_copy(b_hbm, b_v)
    wregs = [w_v[pl.ds(j * LANES, LANES)] for j in range(J)]
    bregs = [b_v[pl.ds(j * LANES, LANES)] for j in range(J)]

    def start_in(c, idxb, ib, sem):
        bs = pl.ds(b0 + c * CB, CB)
        pltpu.sync_copy(x_hbm.at[bs], idxb)
        for bb in range(CB):
            pltpu.async_copy(table_hbm.at[idxb.at[bb]], ib.at[bb], sem)

    def wait_in(ib, sem):
        pltpu.make_async_copy(out_hbm.at[pl.ds(0, CB)], ib, sem).wait()

    def start_out(c, ob, sem):
        pltpu.async_copy(ob, out_hbm.at[pl.ds(b0 + c * CB, CB)], sem)

    def wait_out(ob, sem):
        pltpu.make_async_copy(ob, out_hbm.at[pl.ds(0, CB)], sem).wait()

    def compute(ib, ob):
        for bb in range(CB):
            def ll_body(ll, _):
                v = [ib[bb, ll, pl.ds(j * LANES, LANES)] for j in range(J)]
                s = jnp.sum((v[0] + v[1]) + (v[2] + v[3]))
                s2 = jnp.sum((v[0] * v[0] + v[1] * v[1])
                             + (v[2] * v[2] + v[3] * v[3]))
                mean = s * (1.0 / D)
                var = s2 * (1.0 / D) - mean * mean
                rstd = _rsqrt_s(var + EPS)
                mean_b = jnp.full((LANES,), mean, jnp.float32)
                rstd_b = jnp.full((LANES,), rstd, jnp.float32)
                for j in range(J):
                    ob[bb, ll, pl.ds(j * LANES, LANES)] = (
                        (v[j] - mean_b) * (rstd_b * wregs[j]) + bregs[j])
                return 0

            lax.fori_loop(0, L, ll_body, 0, unroll=2)

    start_in(0, idx0, ib0, si0)

    def c2_body(c2, _):
        c = 2 * c2
        start_in(c + 1, idx1, ib1, si1)
        wait_in(ib0, si0)

        @pl.when(c2 > 0)
        def _():
            wait_out(ob0, so0)

        compute(ib0, ob0)
        start_out(c, ob0, so0)

        @pl.when(c2 < (n_chunks // 2) - 1)
        def _():
            start_in(c + 2, idx0, ib0, si0)

        wait_in(ib1, si1)

        @pl.when(c2 > 0)
        def _():
            wait_out(ob1, so1)

        compute(ib1, ob1)
        start_out(c + 1, ob1, so1)
        return 0

    lax.fori_loop(0, n_chunks // 2, c2_body, 0)
    wait_out(ob0, so0)
    wait_out(ob1, so1)


def kernel(x, table, ln_weight, ln_bias):
    B, Lx = x.shape
    run = pl.kernel(
        _body,
        out_type=jax.ShapeDtypeStruct((B, Lx, D), jnp.float32),
        mesh=plsc.VectorSubcoreMesh(
            core_axis_name="c", subcore_axis_name="s",
            num_cores=NC, num_subcores=NS,
        ),
        scratch_types=[
            pltpu.VMEM((CB, L), jnp.int32),
            pltpu.VMEM((CB, L), jnp.int32),
            pltpu.VMEM((CB, L, D), jnp.float32),
            pltpu.VMEM((CB, L, D), jnp.float32),
            pltpu.VMEM((CB, L, D), jnp.float32),
            pltpu.VMEM((CB, L, D), jnp.float32),
            pltpu.VMEM((D,), jnp.float32),
            pltpu.VMEM((D,), jnp.float32),
            pltpu.SemaphoreType.DMA,
            pltpu.SemaphoreType.DMA,
            pltpu.SemaphoreType.DMA,
            pltpu.SemaphoreType.DMA,
        ],
        compiler_params=pltpu.CompilerParams(
            needs_layout_passes=False, use_tc_tiling_on_sc=False),
    )
    return run(x, table, ln_weight, ln_bias)


# speedup vs baseline: 1.4217x; 1.4217x over previous
import jax
import jax.numpy as jnp
from jax import lax
from jax.experimental import pallas as pl
from jax.experimental.pallas import tpu as pltpu, tpu_sc as plsc

NC, NS, LANES = 2, 16, 16
NW = NC * NS
D = 64
CHUNK = 400
NBUF = 4
EPS = 1e-5
J = D // LANES


def _rsqrt_s(v):
    i = lax.bitcast_convert_type(v, jnp.int32)
    y = lax.bitcast_convert_type(jnp.int32(0x5F3759DF) - (i >> 1), jnp.float32)
    for _ in range(3):
        y = y * (1.5 - 0.5 * v * y * y)
    return y


def _body(x_hbm, table_hbm, w_hbm, b_hbm, out_hbm,
          idxs, bufs, w_v, b_v, sis, sos):
    per_w = x_hbm.shape[0] // NW
    n_chunks = per_w // CHUNK
    wid = lax.axis_index("s") * NC + lax.axis_index("c")
    base = wid * per_w
    pltpu.sync_copy(w_hbm, w_v)
    pltpu.sync_copy(b_hbm, b_v)
    wregs = [w_v[pl.ds(j * LANES, LANES)] for j in range(J)]
    bregs = [b_v[pl.ds(j * LANES, LANES)] for j in range(J)]

    def start_in(c, k):
        rs = pl.ds(base + c * CHUNK, CHUNK)
        pltpu.sync_copy(x_hbm.at[rs], idxs[k])
        pltpu.async_copy(table_hbm.at[idxs[k]], bufs[k], sis[k])

    def wait_in(k):
        pltpu.make_async_copy(out_hbm.at[pl.ds(0, CHUNK)], bufs[k], sis[k]).wait()

    def start_out(c, k):
        pltpu.async_copy(bufs[k], out_hbm.at[pl.ds(base + c * CHUNK, CHUNK)],
                         sos[k])

    def wait_out(k):
        pltpu.make_async_copy(bufs[k], out_hbm.at[pl.ds(0, CHUNK)], sos[k]).wait()

    def compute(k):
        buf = bufs[k]

        def row(r, _):
            v = [buf[r, pl.ds(j * LANES, LANES)] for j in range(J)]
            s = jnp.sum((v[0] + v[1]) + (v[2] + v[3]))
            s2 = jnp.sum((v[0] * v[0] + v[1] * v[1])
                         + (v[2] * v[2] + v[3] * v[3]))
            mean = s * (1.0 / D)
            var = s2 * (1.0 / D) - mean * mean
            rstd = _rsqrt_s(var + EPS)
            mean_b = jnp.full((LANES,), mean, jnp.float32)
            rstd_b = jnp.full((LANES,), rstd, jnp.float32)
            for j in range(J):
                buf[r, pl.ds(j * LANES, LANES)] = (
                    (v[j] - mean_b) * (rstd_b * wregs[j]) + bregs[j])
            return 0

        lax.fori_loop(0, CHUNK, row, 0, unroll=4)

    start_in(0, 0)
    start_in(1, 1)

    def outer(c4, _):
        for k in range(NBUF):
            c = c4 * NBUF + k
            wait_in(k)
            compute(k)
            start_out(c, k)
            kn = (k + 2) % NBUF
            if k >= 2:
                wait_out(kn)
            else:
                @pl.when(c4 > 0)
                def _():
                    wait_out(kn)

            @pl.when(c + 2 < n_chunks)
            def _():
                start_in(c + 2, kn)
        return 0

    lax.fori_loop(0, n_chunks // NBUF, outer, 0)
    wait_out((n_chunks - 2) % NBUF)
    wait_out((n_chunks - 1) % NBUF)


def kernel(x, table, ln_weight, ln_bias):
    B, L = x.shape
    n = B * L

    def body(x_hbm, table_hbm, w_hbm, b_hbm, out_hbm, *scratch):
        idxs = scratch[0:NBUF]
        bufs = scratch[NBUF:2 * NBUF]
        w_v, b_v = scratch[2 * NBUF], scratch[2 * NBUF + 1]
        sis = scratch[2 * NBUF + 2: 2 * NBUF + 2 + NBUF]
        sos = scratch[2 * NBUF + 2 + NBUF:]
        _body(x_hbm, table_hbm, w_hbm, b_hbm, out_hbm,
              idxs, bufs, w_v, b_v, sis, sos)

    run = pl.kernel(
        body,
        out_type=jax.ShapeDtypeStruct((n, D), jnp.float32),
        mesh=plsc.VectorSubcoreMesh(
            core_axis_name="c", subcore_axis_name="s",
            num_cores=NC, num_subcores=NS,
        ),
        scratch_types=(
            [pltpu.VMEM((CHUNK,), jnp.int32)] * NBUF
            + [pltpu.VMEM((CHUNK, D), jnp.float32)] * NBUF
            + [pltpu.VMEM((D,), jnp.float32)] * 2
            + [pltpu.SemaphoreType.DMA] * (2 * NBUF)
        ),
        compiler_params=pltpu.CompilerParams(
            needs_layout_passes=False, use_tc_tiling_on_sc=False),
    )
    out = run(x.reshape(-1), table, ln_weight, ln_bias)
    return out.reshape(B, L, D)
